# baseline (speedup 1.0000x reference)
import functools

import numpy as np

import jax
import jax.numpy as jnp
from jax import lax
from jax.experimental import pallas as pl
from jax.experimental.pallas import tpu as pltpu
from jax.sharding import Mesh, PartitionSpec as P
from jax.experimental.shard_map import shard_map

N = 4096
D = 256
BM1 = 256
BM2 = 512


def _k1_body(adjh_ref, x_ref, w1_ref, wn_ref, b_ref,
             abf_ref, xw1_ref, xw0_ref):
    i = pl.program_id(0)

    @pl.when(i == 0)
    def _():
        xw0_ref[...] = jnp.dot(
            x_ref[...], w1_ref[...], preferred_element_type=jnp.float32
        ).astype(jnp.bfloat16)

    ab = adjh_ref[...].astype(jnp.bfloat16)
    abf_ref[...] = ab
    acc = jnp.dot(ab, xw0_ref[...], preferred_element_type=jnp.float32)
    h = jnp.maximum(acc + b_ref[...], 0.0).astype(jnp.bfloat16)
    xw1_ref[...] = jnp.dot(
        h, wn_ref[...], preferred_element_type=jnp.float32
    ).astype(jnp.bfloat16)


def _k1(adj_half, xbf, w1, wn, b):
    m = adj_half.shape[0]
    return pl.pallas_call(
        _k1_body,
        grid=(m // BM1,),
        in_specs=[
            pl.BlockSpec((BM1, N), lambda i: (i, 0)),
            pl.BlockSpec((N, D), lambda i: (0, 0)),
            pl.BlockSpec((D, D), lambda i: (0, 0)),
            pl.BlockSpec((D, D), lambda i: (0, 0)),
            pl.BlockSpec((1, D), lambda i: (0, 0)),
        ],
        out_specs=(
            pl.BlockSpec((BM1, N), lambda i: (i, 0)),
            pl.BlockSpec((BM1, D), lambda i: (i, 0)),
        ),
        out_shape=(
            jax.ShapeDtypeStruct((m, N), jnp.bfloat16),
            jax.ShapeDtypeStruct((m, D), jnp.bfloat16),
        ),
        scratch_shapes=[pltpu.VMEM((N, D), jnp.bfloat16)],
        compiler_params=pltpu.CompilerParams(
            dimension_semantics=("arbitrary",),
        ),
    )(adj_half, xbf, w1, wn, b)


def _kl_body(abf_ref, xw_ref, wn_ref, b_ref, out_ref, *, fuse_next):
    acc = jnp.dot(abf_ref[...], xw_ref[...], preferred_element_type=jnp.float32)
    h = jnp.maximum(acc + b_ref[...], 0.0)
    if fuse_next:
        out_ref[...] = jnp.dot(
            h.astype(jnp.bfloat16), wn_ref[...],
            preferred_element_type=jnp.float32,
        ).astype(jnp.bfloat16)
    else:
        out_ref[...] = h


def _kl(abf_half, xw, wn, b, fuse_next):
    m = abf_half.shape[0]
    odt = jnp.bfloat16 if fuse_next else jnp.float32
    return pl.pallas_call(
        functools.partial(_kl_body, fuse_next=fuse_next),
        grid=(m // BM2,),
        in_specs=[
            pl.BlockSpec((BM2, N), lambda i: (i, 0)),
            pl.BlockSpec((N, D), lambda i: (0, 0)),
            pl.BlockSpec((D, D), lambda i: (0, 0)),
            pl.BlockSpec((1, D), lambda i: (0, 0)),
        ],
        out_specs=pl.BlockSpec((BM2, D), lambda i: (i, 0)),
        out_shape=jax.ShapeDtypeStruct((m, D), odt),
        compiler_params=pltpu.CompilerParams(
            dimension_semantics=("parallel",),
        ),
    )(abf_half, xw, wn, b)


def _forward(adj_half, xbf, w1, w2, w3, b1, b2, b3, axis_name):
    abf, xw1_half = _k1(adj_half, xbf, w1, w2, b1)
    xw1 = (lax.all_gather(xw1_half, axis_name, axis=0, tiled=True)
           if axis_name else xw1_half)
    xw2_half = _kl(abf, xw1, w3, b2, True)
    xw2 = (lax.all_gather(xw2_half, axis_name, axis=0, tiled=True)
           if axis_name else xw2_half)
    return _kl(abf, xw2, w3, b3, False)


@jax.jit
def kernel(x, adj, W1, b1, W2, b2, W3, b3):
    xbf = x.astype(jnp.bfloat16)
    w1 = W1.astype(jnp.bfloat16)
    w2 = W2.astype(jnp.bfloat16)
    w3 = W3.astype(jnp.bfloat16)
    b1r = b1.reshape(1, D)
    b2r = b2.reshape(1, D)
    b3r = b3.reshape(1, D)

    devs = jax.devices()
    if len(devs) >= 2:
        mesh = Mesh(np.array(devs[:2]), ("r",))
        f = shard_map(
            functools.partial(_forward, axis_name="r"),
            mesh=mesh,
            in_specs=(P("r", None), P(), P(), P(), P(), P(), P(), P()),
            out_specs=P("r", None),
            check_rep=False,
        )
    else:
        f = functools.partial(_forward, axis_name=None)
    return f(adj, xbf, w1, w2, w3, b1r, b2r, b3r)


# speedup vs baseline: 8.6576x; 8.6576x over previous
import jax
import jax.numpy as jnp
from jax.experimental import pallas as pl
from jax.experimental.pallas import tpu as pltpu

N = 4096
D = 256
BM = 512
I = N // BM
H = BM // 2


def _body(x_ref, adj_ref, w1_ref, wn_ref, b_ref, out_ref,
          adjbf_ref, xwa_ref, xwb_ref):
    p = pl.program_id(0)
    i = pl.program_id(1)
    r = pl.ds(i * BM, BM)

    @pl.when((p == 0) & (i == 0))
    def _():
        xwb_ref[...] = jnp.dot(
            x_ref[...], w1_ref[...], preferred_element_type=jnp.float32
        ).astype(jnp.bfloat16)

    @pl.when(p == 0)
    def _():
        ab = adj_ref[...].astype(jnp.bfloat16)
        adjbf_ref[r, :] = ab
        acc = jnp.dot(ab, xwb_ref[...], preferred_element_type=jnp.float32)
        h = jnp.maximum(acc + b_ref[0], 0.0).astype(jnp.bfloat16)
        xwa_ref[r, :] = jnp.dot(
            h, wn_ref[0], preferred_element_type=jnp.float32
        ).astype(jnp.bfloat16)

    @pl.when(p == 1)
    def _():
        for s in range(2):
            rs = pl.ds(i * BM + s * H, H)
            acc = jnp.dot(
                adjbf_ref[rs, :], xwa_ref[...],
                preferred_element_type=jnp.float32,
            )
            h = jnp.maximum(acc + b_ref[1], 0.0).astype(jnp.bfloat16)
            xwb_ref[rs, :] = jnp.dot(
                h, wn_ref[1], preferred_element_type=jnp.float32
            ).astype(jnp.bfloat16)

    @pl.when(p == 2)
    def _():
        for s in range(2):
            rs = pl.ds(i * BM + s * H, H)
            acc = jnp.dot(
                adjbf_ref[rs, :], xwb_ref[...],
                preferred_element_type=jnp.float32,
            )
            out_ref[pl.ds(s * H, H), :] = jnp.maximum(acc + b_ref[2], 0.0)


@jax.jit
def kernel(x, adj, W1, b1, W2, b2, W3, b3):
    xbf = x.astype(jnp.bfloat16)
    w1 = W1.astype(jnp.bfloat16)
    wn = jnp.stack([W2, W3]).astype(jnp.bfloat16)
    b = jnp.stack([b1, b2, b3]).reshape(3, 1, D)

    last = I - 1
    return pl.pallas_call(
        _body,
        grid=(3, I),
        in_specs=[
            pl.BlockSpec((N, D), lambda p, i: (0, 0)),
            pl.BlockSpec((BM, N), lambda p, i: (jnp.where(p == 0, i, last), 0)),
            pl.BlockSpec((D, D), lambda p, i: (0, 0)),
            pl.BlockSpec((2, D, D), lambda p, i: (0, 0, 0)),
            pl.BlockSpec((3, 1, D), lambda p, i: (0, 0, 0)),
        ],
        out_specs=pl.BlockSpec((BM, D), lambda p, i: (jnp.where(p == 2, i, 0), 0)),
        out_shape=jax.ShapeDtypeStruct((N, D), jnp.float32),
        scratch_shapes=[
            pltpu.VMEM((N, N), jnp.bfloat16),
            pltpu.VMEM((N, D), jnp.bfloat16),
            pltpu.VMEM((N, D), jnp.bfloat16),
        ],
        compiler_params=pltpu.CompilerParams(
            dimension_semantics=("arbitrary", "arbitrary"),
        ),
    )(xbf, adj, w1, wn, b)


# speedup vs baseline: 10.2921x; 1.1888x over previous
import jax
import jax.numpy as jnp
from jax.experimental import pallas as pl
from jax.experimental.pallas import tpu as pltpu

N = 4096
D = 256
BM = 512
I = N // BM
B2 = 2 * BM


def _body(x_ref, adj_ref, w1_ref, wn_ref, b_ref, out_ref,
          adjbf_ref, xwa_ref, xwb_ref):
    p = pl.program_id(0)
    i = pl.program_id(1)
    r = pl.ds(i * BM, BM)

    @pl.when((p == 0) & (i == 0))
    def _():
        xwb_ref[...] = jnp.dot(
            x_ref[...], w1_ref[...], preferred_element_type=jnp.float32
        ).astype(jnp.bfloat16)

    @pl.when(p == 0)
    def _():
        ab = adj_ref[...].astype(jnp.bfloat16)
        adjbf_ref[r, :] = ab
        acc = jnp.dot(ab, xwb_ref[...], preferred_element_type=jnp.float32)
        h = jnp.maximum(acc + b_ref[0], 0.0).astype(jnp.bfloat16)
        xwa_ref[r, :] = jnp.dot(
            h, wn_ref[0], preferred_element_type=jnp.float32
        ).astype(jnp.bfloat16)

    r2 = pl.ds(i * BM, B2)

    @pl.when((p == 1) & (i % 2 == 0))
    def _():
        acc = jnp.dot(
            adjbf_ref[r2, :], xwa_ref[...], preferred_element_type=jnp.float32
        )
        h = jnp.maximum(acc + b_ref[1], 0.0).astype(jnp.bfloat16)
        xwb_ref[r2, :] = jnp.dot(
            h, wn_ref[1], preferred_element_type=jnp.float32
        ).astype(jnp.bfloat16)

    @pl.when((p == 2) & (i % 2 == 0))
    def _():
        acc = jnp.dot(
            adjbf_ref[r2, :], xwb_ref[...], preferred_element_type=jnp.float32
        )
        out_ref[...] = jnp.maximum(acc + b_ref[2], 0.0)


@jax.jit
def kernel(x, adj, W1, b1, W2, b2, W3, b3):
    xbf = x.astype(jnp.bfloat16)
    w1 = W1.astype(jnp.bfloat16)
    wn = jnp.stack([W2, W3]).astype(jnp.bfloat16)
    b = jnp.stack([b1, b2, b3]).reshape(3, 1, D)

    last = I - 1
    return pl.pallas_call(
        _body,
        grid=(3, I),
        in_specs=[
            pl.BlockSpec((N, D), lambda p, i: (0, 0)),
            pl.BlockSpec((BM, N), lambda p, i: (jnp.where(p == 0, i, last), 0)),
            pl.BlockSpec((D, D), lambda p, i: (0, 0)),
            pl.BlockSpec((2, D, D), lambda p, i: (0, 0, 0)),
            pl.BlockSpec((3, 1, D), lambda p, i: (0, 0, 0)),
        ],
        out_specs=pl.BlockSpec(
            (B2, D), lambda p, i: (jnp.where(p == 2, i // 2, 0), 0)
        ),
        out_shape=jax.ShapeDtypeStruct((N, D), jnp.float32),
        scratch_shapes=[
            pltpu.VMEM((N, N), jnp.bfloat16),
            pltpu.VMEM((N, D), jnp.bfloat16),
            pltpu.VMEM((N, D), jnp.bfloat16),
        ],
        compiler_params=pltpu.CompilerParams(
            dimension_semantics=("arbitrary", "arbitrary"),
        ),
    )(xbf, adj, w1, wn, b)
